# baseline (speedup 1.0000x reference)
import functools

import jax
import jax.numpy as jnp
from jax import lax
from jax.experimental import pallas as pl
from jax.experimental.pallas import tpu as pltpu
from jax.experimental.pallas import tpu_sc as plsc

NC, NS = 2, 16
NW = NC * NS
BATCH = 16384
D = 64
NODE_TILES = 125000
B_PER_W = BATCH // NW
K = 16
CH = B_PER_W // K


def kernel(inp, embedding, context_embedding):
    idx = inp.astype(jnp.int32)
    emb3 = embedding.reshape(NODE_TILES, 8, D)
    ctx3 = context_embedding.reshape(NODE_TILES, 8, D)
    mesh = plsc.VectorSubcoreMesh(
        core_axis_name="c", subcore_axis_name="s", num_cores=NC, num_subcores=NS
    )

    @functools.partial(
        pl.kernel,
        out_type=jax.ShapeDtypeStruct((BATCH, 2 * D), jnp.float32),
        mesh=mesh,
        scratch_types=[
            pltpu.VMEM((B_PER_W,), jnp.int32),
            pltpu.VMEM((K, 8, D), jnp.float32),
            pltpu.VMEM((K, 8, D), jnp.float32),
            pltpu.VMEM((K, 8, D), jnp.float32),
            pltpu.VMEM((K, 8, D), jnp.float32),
            pltpu.VMEM((B_PER_W // 2, 2 * D), jnp.float32),
            pltpu.SemaphoreType.DMA,
            pltpu.SemaphoreType.DMA,
            pltpu.SemaphoreType.DMA,
            pltpu.SemaphoreType.DMA,
        ],
        compiler_params=pltpu.CompilerParams(needs_layout_passes=False),
    )
    def _gather2(idx_hbm, emb_hbm, ctx_hbm, out_hbm,
                 idx_v, se_a, sc_a, se_b, sc_b, cat_v,
                 sem_ea, sem_ca, sem_eb, sem_cb):
        wid = lax.axis_index("s") * NC + lax.axis_index("c")
        base = wid * B_PER_W
        pltpu.sync_copy(idx_hbm.at[pl.ds(base, B_PER_W)], idx_v)
        lanes = lax.iota(jnp.int32, K)

        def issue(n, se, sc, sem_e, sem_c):
            s = idx_v[pl.ds(n * K, K)]
            tvec = lax.shift_right_logical(s, 3)
            for j in range(K):
                tj = tvec[j]
                pltpu.make_async_copy(emb_hbm.at[tj], se.at[j], sem_e).start()
                pltpu.make_async_copy(ctx_hbm.at[tj], sc.at[j], sem_c).start()

        def drain_extract(n, se, sc, sem_e, sem_c):
            for j in range(K):
                pltpu.make_async_copy(emb_hbm.at[0], se.at[j], sem_e).wait()
                pltpu.make_async_copy(ctx_hbm.at[0], sc.at[j], sem_c).wait()
            s = idx_v[pl.ds(n * K, K)]
            rvec = lax.bitwise_and(s, 7)
            rows = lax.bitwise_and(n * K, B_PER_W // 2 - 1) + lanes
            for q in range(D):
                qv = jnp.full((K,), q, jnp.int32)
                ve = plsc.load_gather(se, [lanes, rvec, qv])
                plsc.store_scatter(cat_v, [rows, qv], ve)
                vc = plsc.load_gather(sc, [lanes, rvec, qv])
                plsc.store_scatter(cat_v, [rows, qv + D], vc)

        issue(0, se_a, sc_a, sem_ea, sem_ca)

        def body(i, carry):
            n0 = 2 * i
            n1 = n0 + 1

            @pl.when(n1 < CH)
            def _():
                issue(n1, se_b, sc_b, sem_eb, sem_cb)

            drain_extract(n0, se_a, sc_a, sem_ea, sem_ca)

            @pl.when(n1 + 1 < CH)
            def _():
                issue(n1 + 1, se_a, sc_a, sem_ea, sem_ca)

            @pl.when(n1 < CH)
            def _():
                drain_extract(n1, se_b, sc_b, sem_eb, sem_cb)

            @pl.when(n1 == CH // 2 - 1)
            def _():
                pltpu.sync_copy(cat_v, out_hbm.at[pl.ds(base, B_PER_W // 2), :])

            return carry

        lax.fori_loop(0, (CH + 1) // 2, body, 0)
        pltpu.sync_copy(
            cat_v, out_hbm.at[pl.ds(base + B_PER_W // 2, B_PER_W // 2), :])

    return _gather2(idx, emb3, ctx3)
